# Initial kernel scaffold; baseline (speedup 1.0000x reference)
import jax
import jax.numpy as jnp
from jax.experimental import pallas as pl


def kernel(x, edge_index, edge_weight, categories_value, params):
    raise NotImplementedError("write your pallas kernel here")


# speedup vs baseline: 1.0268x; 1.0268x over previous
import functools

import jax
import jax.numpy as jnp
from jax.experimental import pallas as pl
from jax.experimental.pallas import tpu as pltpu

N = 50000
E = 800000
F_NUM = 16
HID = 32
IDE = 16
EMB = 8
NCAT = 3
K = 3
NCLS = 2
EPS = 1e-5

BN = 400
GRID = N // BN


def _elu(x):
    return jnp.where(x > 0, x, jnp.exp(jnp.minimum(x, 0.0)) - 1.0)


def _enc_body(x_ref, idr_ref, e0_ref, e1_ref, e2_ref, dega_ref, degb_ref,
              w0_ref, b0_ref, wid_ref, bid_ref, wemb_ref, bemb_ref,
              lng_ref, lnb_ref, tw_ref,
              g0_ref, g1_ref, g2_ref, g3_ref, dinv_ref):
    h0 = _elu(jnp.dot(x_ref[...], w0_ref[...],
                      preferred_element_type=jnp.float32) + b0_ref[...])
    ide = _elu(jnp.dot(idr_ref[...], wid_ref[...],
                       preferred_element_type=jnp.float32) + bid_ref[...])
    cat = jnp.concatenate([e0_ref[...], e1_ref[...], e2_ref[...]], axis=1)
    ee = _elu(jnp.dot(cat, wemb_ref[...],
                      preferred_element_type=jnp.float32) + bemb_ref[...])
    h = jnp.concatenate([ide, h0, ee], axis=1)
    mu = jnp.mean(h, axis=-1, keepdims=True)
    var = jnp.mean((h - mu) ** 2, axis=-1, keepdims=True)
    hn = (h - mu) * jax.lax.rsqrt(var + EPS) * lng_ref[...] + lnb_ref[...]
    tw = tw_ref[...]
    g0_ref[...] = jnp.dot(hn, tw[0], preferred_element_type=jnp.float32)
    g1_ref[...] = jnp.dot(hn, tw[1], preferred_element_type=jnp.float32)
    g2_ref[...] = jnp.dot(hn, tw[2], preferred_element_type=jnp.float32)
    g3_ref[...] = jnp.dot(hn, tw[3], preferred_element_type=jnp.float32)
    deg = dega_ref[...] + degb_ref[...]
    dinv_ref[...] = jnp.where(
        deg > 0, jax.lax.rsqrt(jnp.maximum(deg, 1e-30)), 0.0)


def _encoder(x, idr, e0, e1, e2, dega, degb, params):
    D = IDE + HID + EMB * NCAT
    full = lambda shape: pl.BlockSpec(shape, lambda i: tuple(0 for _ in shape))
    row = lambda w: pl.BlockSpec((BN, w), lambda i: (i, 0))
    out_shapes = [jax.ShapeDtypeStruct((N, HID), jnp.float32) for _ in range(4)]
    out_shapes.append(jax.ShapeDtypeStruct((N, 1), jnp.float32))
    out_specs = [row(HID) for _ in range(4)] + [row(1)]
    return pl.pallas_call(
        _enc_body,
        grid=(GRID,),
        in_specs=[row(F_NUM), row(IDE), row(EMB), row(EMB), row(EMB),
                  row(1), row(1),
                  full((F_NUM, HID)), full((HID,)),
                  full((IDE, IDE)), full((IDE,)),
                  full((EMB * NCAT, EMB * NCAT)), full((EMB * NCAT,)),
                  full((D,)), full((D,)),
                  full((K + 1, D, HID))],
        out_specs=out_specs,
        out_shape=out_shapes,
    )(x, idr, e0, e1, e2, dega, degb,
      params['W0'], params['b0'], params['W_id'], params['b_id'],
      params['W_emb'], params['b_emb'], params['ln0_g'], params['ln0_b'],
      params['tag_W'])


def _head_body(t_ref, tb_ref, lng_ref, lnb_ref, w1_ref, b1_ref, out_ref):
    t = jnp.maximum(t_ref[...] + tb_ref[...], 0.0)
    mu = jnp.mean(t, axis=-1, keepdims=True)
    var = jnp.mean((t - mu) ** 2, axis=-1, keepdims=True)
    tn = (t - mu) * jax.lax.rsqrt(var + EPS) * lng_ref[...] + lnb_ref[...]
    logits = jnp.dot(tn, w1_ref[...], preferred_element_type=jnp.float32) + b1_ref[...]
    m = jnp.max(logits, axis=-1, keepdims=True)
    lse = m + jnp.log(jnp.sum(jnp.exp(logits - m), axis=-1, keepdims=True))
    out_ref[...] = logits - lse


def _head(t, params):
    full = lambda shape: pl.BlockSpec(shape, lambda i: tuple(0 for _ in shape))
    row = lambda w: pl.BlockSpec((BN, w), lambda i: (i, 0))
    return pl.pallas_call(
        _head_body,
        grid=(GRID,),
        in_specs=[row(HID), full((HID,)), full((HID,)), full((HID,)),
                  full((HID, NCLS)), full((NCLS,))],
        out_specs=row(NCLS),
        out_shape=jax.ShapeDtypeStruct((N, NCLS), jnp.float32),
    )(t, params['tag_b'], params['ln1_g'], params['ln1_b'],
      params['W1'], params['b1'])


def kernel(x, edge_index, edge_weight, categories_value, params):
    src = edge_index[0]
    dst = edge_index[1]
    deg = jnp.zeros((N,), jnp.float32).at[dst].add(edge_weight)
    dega = deg[:, None]
    degb = jnp.zeros_like(dega)
    idr = jnp.take(params['id_table'], categories_value[:, 0], axis=0)
    e0 = jnp.take(params['emb_tables'][0], categories_value[:, 1], axis=0)
    e1 = jnp.take(params['emb_tables'][1], categories_value[:, 2], axis=0)
    e2 = jnp.take(params['emb_tables'][2], categories_value[:, 3], axis=0)

    g0, g1, g2, g3, dinv2 = _encoder(x, idr, e0, e1, e2, dega, degb, params)
    dinv = dinv2[:, 0]

    norm = dinv[src] * edge_weight * dinv[dst]
    t = g3
    for g_k in (g2, g1, g0):
        msg = norm[:, None] * jnp.take(t, src, axis=0)
        t = jnp.zeros_like(t).at[dst].add(msg) + g_k

    return _head(t, params)


# speedup vs baseline: 3.5659x; 3.4728x over previous
import functools

import jax
import jax.numpy as jnp
from jax import lax
from jax.experimental import pallas as pl
from jax.experimental.pallas import tpu as pltpu
from jax.experimental.pallas import tpu_sc as plsc

N = 50000
E = 800000
F_NUM = 16
HID = 32
IDE = 16
EMB = 8
NCAT = 3
K = 3
NCLS = 2
EPS = 1e-5

NC = 2
NS = 16
L = 16

CH = 128
EC = E // NC
NCHUNK_C = EC // CH
ROWS_T = N // NS

BN = 400
GRID = N // BN

_mesh = plsc.VectorSubcoreMesh(core_axis_name="c", subcore_axis_name="s")


def _i16():
    return jax.lax.iota(jnp.int32, 16)


DEGC = 128
NDEGF = N // DEGC
DEGR = N - NDEGF * DEGC
NPAD = (NDEGF + 1) * DEGC
NPC = NPAD // DEGC
EMBC = 128
NEMBC = (N + EMBC - 1) // EMBC
TAIL_OFF = N - EMBC


def _sc_pre_body(dst_hbm, w_hbm, cv_hbm, idt_hbm, emb_hbm,
                 degp_hbm, idr_hbm, er_hbm,
                 deg_sh, zb_v, dstd_v, wd_v, cv_v, idx_v, idrows_v, erows_v,
                 gsem, osem):
    c = lax.axis_index("c")
    s = lax.axis_index("s")
    w = c * NS + s

    @pl.loop(0, DEGC // L)
    def _z(i):
        zb_v[pl.ds(i * L, L)] = jnp.zeros((L,), jnp.float32)

    nzc = jnp.where(s < NPC % NS, NPC // NS + 1, NPC // NS)

    @pl.loop(0, nzc)
    def _zd(k):
        cid = s + NS * k
        pltpu.sync_copy(zb_v, deg_sh.at[pl.ds(cid * DEGC, DEGC)])

    plsc.subcore_barrier()

    nec = jnp.where(s < 5, NCHUNK_C // NS + 1, NCHUNK_C // NS)

    @pl.loop(0, nec)
    def _e(k):
        cid = s + NS * k
        off = c * EC + cid * CH
        pltpu.sync_copy(dst_hbm.at[pl.ds(off, CH)], dstd_v.at[0])
        pltpu.sync_copy(w_hbm.at[pl.ds(off, CH)], wd_v)
        pltpu.sync_copy(wd_v, deg_sh.at[dstd_v.at[0]], add=True)

    plsc.subcore_barrier()

    @pl.loop(0, nzc)
    def _wd(k):
        cid = s + NS * k
        pltpu.sync_copy(deg_sh.at[pl.ds(cid * DEGC, DEGC)],
                        degp_hbm.at[c, 0, pl.ds(cid * DEGC, DEGC)])

    ngc = jnp.where(w < NEMBC % (NC * NS),
                    NEMBC // (NC * NS) + 1, NEMBC // (NC * NS))

    @pl.loop(0, ngc)
    def _g(k):
        cid = w + NC * NS * k
        off = jnp.where(cid == NEMBC - 1, TAIL_OFF, cid * EMBC)
        off = pl.multiple_of(off, 8)
        pltpu.sync_copy(cv_hbm.at[pl.ds(off * 4, EMBC * 4)], cv_v)
        for t in range(4):
            for i in range(EMBC // L):
                flat = (jnp.full((L,), i * L, jnp.int32) + _i16()) * 4 + t
                v = plsc.load_gather(cv_v, [flat])
                if t > 0:
                    v = v + jnp.full((L,), (t - 1) * N, jnp.int32)
                idx_v[t, pl.ds(i * L, L)] = v
        cp1 = pltpu.async_copy(idt_hbm.at[idx_v.at[0]], idrows_v, gsem)
        cp2 = pltpu.async_copy(emb_hbm.at[idx_v.at[1]], erows_v.at[0], gsem)
        cp3 = pltpu.async_copy(emb_hbm.at[idx_v.at[2]], erows_v.at[1], gsem)
        cp4 = pltpu.async_copy(emb_hbm.at[idx_v.at[3]], erows_v.at[2], gsem)
        cp1.wait()
        cp2.wait()
        cp3.wait()
        cp4.wait()
        o1 = pltpu.async_copy(idrows_v, idr_hbm.at[pl.ds(off, EMBC)], osem)
        o2 = pltpu.async_copy(erows_v.at[0], er_hbm.at[0, pl.ds(off, EMBC)],
                              osem)
        o3 = pltpu.async_copy(erows_v.at[1], er_hbm.at[1, pl.ds(off, EMBC)],
                              osem)
        o4 = pltpu.async_copy(erows_v.at[2], er_hbm.at[2, pl.ds(off, EMBC)],
                              osem)
        o1.wait()
        o2.wait()
        o3.wait()
        o4.wait()


@functools.partial(
    pl.kernel,
    out_type=[
        jax.ShapeDtypeStruct((NC, 1, NPAD), jnp.float32),
        jax.ShapeDtypeStruct((N, IDE), jnp.float32),
        jax.ShapeDtypeStruct((NCAT, N, EMB), jnp.float32),
    ],
    mesh=_mesh,
    scratch_types=[
        pltpu.VMEM_SHARED((NPAD,), jnp.float32),
        pltpu.VMEM((DEGC,), jnp.float32),
        pltpu.VMEM((1, CH), jnp.int32),
        pltpu.VMEM((CH,), jnp.float32),
        pltpu.VMEM((EMBC * 4,), jnp.int32),
        pltpu.VMEM((4, EMBC), jnp.int32),
        pltpu.VMEM((EMBC, IDE), jnp.float32),
        pltpu.VMEM((NCAT, EMBC, EMB), jnp.float32),
        pltpu.SemaphoreType.DMA,
        pltpu.SemaphoreType.DMA,
    ],
    compiler_params=pltpu.CompilerParams(needs_layout_passes=False, use_tc_tiling_on_sc=False),
)
def _sc_pre(dst_hbm, w_hbm, cv_hbm, idt_hbm, emb_hbm, *rest):
    _sc_pre_body(dst_hbm, w_hbm, cv_hbm, idt_hbm, emb_hbm, *rest)


def _sc_norm_body(src_hbm, dst_hbm, w_hbm, dinv_hbm, norm_hbm,
                  sidx_v, didx_v, wn_v, ds_v, dd_v, nrm_v, gsem):
    c = lax.axis_index("c")
    s = lax.axis_index("s")
    nec = jnp.where(s < NCHUNK_C % NS, NCHUNK_C // NS + 1, NCHUNK_C // NS)

    @pl.loop(0, nec)
    def _e(k):
        cid = s + NS * k
        off = c * EC + cid * CH
        pltpu.sync_copy(src_hbm.at[pl.ds(off, CH)], sidx_v.at[0])
        pltpu.sync_copy(dst_hbm.at[pl.ds(off, CH)], didx_v.at[0])
        pltpu.sync_copy(w_hbm.at[pl.ds(off, CH)], wn_v)
        c1 = pltpu.async_copy(dinv_hbm.at[sidx_v.at[0]], ds_v, gsem)
        c2 = pltpu.async_copy(dinv_hbm.at[didx_v.at[0]], dd_v, gsem)
        c1.wait()
        c2.wait()
        for i in range(CH // L):
            sl = pl.ds(i * L, L)
            nrm_v[sl] = ds_v[sl] * wn_v[sl] * dd_v[sl]
        pltpu.sync_copy(nrm_v, norm_hbm.at[pl.ds(off, CH)])


@functools.partial(
    pl.kernel,
    out_type=jax.ShapeDtypeStruct((E,), jnp.float32),
    mesh=_mesh,
    scratch_types=[
        pltpu.VMEM((1, CH), jnp.int32),
        pltpu.VMEM((1, CH), jnp.int32),
        pltpu.VMEM((CH,), jnp.float32),
        pltpu.VMEM((CH,), jnp.float32),
        pltpu.VMEM((CH,), jnp.float32),
        pltpu.VMEM((CH,), jnp.float32),
        pltpu.SemaphoreType.DMA,
    ],
    compiler_params=pltpu.CompilerParams(needs_layout_passes=False, use_tc_tiling_on_sc=False),
)
def _sc_norm(src_hbm, dst_hbm, w_hbm, dinv_hbm, *rest):
    _sc_norm_body(src_hbm, dst_hbm, w_hbm, dinv_hbm, *rest)


ZB = 4096


def _sc_hop_body(*args):
    tcols = args[:HID]
    src_hbm, dstl_hbm, norm_hbm = args[HID:HID + 3]
    parts = args[HID + 3:2 * HID + 3]
    accs = args[2 * HID + 3:3 * HID + 3]
    src_v, dst_v, norm_v, col_v, gsem, osem = args[3 * HID + 3:]
    c = lax.axis_index("c")
    s = lax.axis_index("s")

    @pl.loop(0, ZB // L)
    def _z(i):
        col_v[pl.ds(i * L, L)] = jnp.zeros((L,), jnp.float32)

    ZT = NPAD - (NPAD // ZB) * ZB

    def _zero_plane(acc):
        @pl.loop(0, NPAD // ZB)
        def _zp(i):
            pltpu.sync_copy(col_v, acc.at[pl.ds(i * ZB, ZB)])
        pltpu.sync_copy(col_v.at[pl.ds(0, ZT)],
                        acc.at[pl.ds((NPAD // ZB) * ZB, ZT)])

    for j in range(HID):
        @pl.when(s == j // 2)
        def _dz(acc=accs[j]):
            _zero_plane(acc)

    plsc.subcore_barrier()

    nec = jnp.where(s < NCHUNK_C % NS, NCHUNK_C // NS + 1, NCHUNK_C // NS)

    @pl.loop(0, nec)
    def _e(k):
        cid = s + NS * k
        off = c * EC + cid * CH
        pltpu.sync_copy(src_hbm.at[pl.ds(off, CH)], src_v.at[0])
        pltpu.sync_copy(dstl_hbm.at[pl.ds(off, CH)], dst_v.at[0])
        pltpu.sync_copy(norm_hbm.at[pl.ds(off, CH)], norm_v)
        gps = [pltpu.async_copy(tcols[j].at[src_v.at[0]],
                                col_v.at[pl.ds(j * CH, CH)], gsem)
               for j in range(HID)]
        for gp in gps:
            gp.wait()
        for g in range(CH // L):
            sl = pl.ds(g * L, L)
            nv = norm_v[sl]
            for j in range(HID):
                sl2 = pl.ds(j * CH + g * L, L)
                col_v[sl2] = col_v[sl2] * nv
        cps = [pltpu.async_copy(col_v.at[pl.ds(j * CH, CH)],
                                accs[j].at[dst_v.at[0]], osem, add=True)
               for j in range(HID)]
        for cp in cps:
            cp.wait()

    plsc.subcore_barrier()

    for j in range(HID):
        @pl.when(s == j // 2)
        def _dw(acc=accs[j], part=parts[j]):
            pltpu.sync_copy(acc, part.at[c, 0])


@functools.partial(
    pl.kernel,
    out_type=[jax.ShapeDtypeStruct((NC, 1, NPAD), jnp.float32)
              for _ in range(HID)],
    mesh=_mesh,
    scratch_types=(
        [pltpu.VMEM_SHARED((NPAD,), jnp.float32) for _ in range(HID)] + [
            pltpu.VMEM((1, CH), jnp.int32),
            pltpu.VMEM((1, CH), jnp.int32),
            pltpu.VMEM((CH,), jnp.float32),
            pltpu.VMEM((ZB,), jnp.float32),
            pltpu.SemaphoreType.DMA,
            pltpu.SemaphoreType.DMA,
        ]),
    compiler_params=pltpu.CompilerParams(needs_layout_passes=False, use_tc_tiling_on_sc=False),
)
def _sc_hop(*args):
    _sc_hop_body(*args)


def _elu(x):
    return jnp.where(x > 0, x, jnp.exp(jnp.minimum(x, 0.0)) - 1.0)


def _enc_body(x_ref, idr_ref, e0_ref, e1_ref, e2_ref, dega_ref, degb_ref,
              w0_ref, b0_ref, wid_ref, bid_ref, wemb_ref, bemb_ref,
              lng_ref, lnb_ref, tw_ref,
              g0_ref, g1_ref, g2_ref, g3_ref, dinv_ref):
    h0 = _elu(jnp.dot(x_ref[...], w0_ref[...],
                      preferred_element_type=jnp.float32) + b0_ref[...])
    ide = _elu(jnp.dot(idr_ref[...], wid_ref[...],
                       preferred_element_type=jnp.float32) + bid_ref[...])
    cat = jnp.concatenate([e0_ref[...], e1_ref[...], e2_ref[...]], axis=1)
    ee = _elu(jnp.dot(cat, wemb_ref[...],
                      preferred_element_type=jnp.float32) + bemb_ref[...])
    h = jnp.concatenate([ide, h0, ee], axis=1)
    mu = jnp.mean(h, axis=-1, keepdims=True)
    var = jnp.mean((h - mu) ** 2, axis=-1, keepdims=True)
    hn = (h - mu) * jax.lax.rsqrt(var + EPS) * lng_ref[...] + lnb_ref[...]
    tw = tw_ref[...]
    g0_ref[...] = jnp.dot(hn, tw[0], preferred_element_type=jnp.float32)
    g1_ref[...] = jnp.dot(hn, tw[1], preferred_element_type=jnp.float32)
    g2_ref[...] = jnp.dot(hn, tw[2], preferred_element_type=jnp.float32)
    g3_ref[...] = jnp.dot(hn, tw[3], preferred_element_type=jnp.float32)
    deg = dega_ref[...] + degb_ref[...]
    dinv_ref[...] = jnp.where(
        deg > 0, jax.lax.rsqrt(jnp.maximum(deg, 1e-30)), 0.0)


def _encoder(x, idr, e0, e1, e2, dega, degb, params):
    D = IDE + HID + EMB * NCAT
    full = lambda shape: pl.BlockSpec(shape, lambda i: tuple(0 for _ in shape))
    row = lambda w: pl.BlockSpec((BN, w), lambda i: (i, 0))
    out_shapes = [jax.ShapeDtypeStruct((N, HID), jnp.float32) for _ in range(4)]
    out_shapes.append(jax.ShapeDtypeStruct((N, 1), jnp.float32))
    out_specs = [row(HID) for _ in range(4)] + [row(1)]
    return pl.pallas_call(
        _enc_body,
        grid=(GRID,),
        in_specs=[row(F_NUM), row(IDE), row(EMB), row(EMB), row(EMB),
                  row(1), row(1),
                  full((F_NUM, HID)), full((HID,)),
                  full((IDE, IDE)), full((IDE,)),
                  full((EMB * NCAT, EMB * NCAT)), full((EMB * NCAT,)),
                  full((D,)), full((D,)),
                  full((K + 1, D, HID))],
        out_specs=out_specs,
        out_shape=out_shapes,
    )(x, idr, e0, e1, e2, dega, degb,
      params['W0'], params['b0'], params['W_id'], params['b_id'],
      params['W_emb'], params['b_emb'], params['ln0_g'], params['ln0_b'],
      params['tag_W'])


CGRID = NPAD // CH


def _comb_body(*refs):
    p_refs = refs[:HID]
    gc_ref = refs[HID]
    out_refs = refs[HID + 1:]
    for j in range(HID):
        out_refs[j][...] = (p_refs[j][0, 0] + p_refs[j][1, 0]
                            + gc_ref[j, 0, :])


def _combine(parts, gcols):
    pspec = pl.BlockSpec((NC, 1, CH), lambda i: (0, 0, i))
    return pl.pallas_call(
        _comb_body,
        grid=(CGRID,),
        in_specs=[pspec] * HID + [pl.BlockSpec((HID, 1, CH),
                                               lambda i: (0, 0, i))],
        out_specs=[pl.BlockSpec((CH,), lambda i: (i,))] * HID,
        out_shape=[jax.ShapeDtypeStruct((NPAD,), jnp.float32)] * HID,
    )(*parts, gcols)


def _head_body(*refs):
    p_refs = refs[:HID]
    (gc_ref, tb_ref, lng_ref, lnb_ref, w1t_ref, b1_ref) = refs[HID:HID + 6]
    out_ref = refs[HID + 6]
    t = jnp.concatenate(
        [p_refs[j][...].sum(axis=0) for j in range(HID)], axis=0
    ) + gc_ref[:, 0, :]
    t = jnp.maximum(t + tb_ref[...][:, None], 0.0)
    mu = jnp.mean(t, axis=0, keepdims=True)
    var = jnp.mean((t - mu) ** 2, axis=0, keepdims=True)
    tn = ((t - mu) * jax.lax.rsqrt(var + EPS) * lng_ref[...][:, None]
          + lnb_ref[...][:, None])
    logits = jnp.dot(w1t_ref[...], tn,
                     preferred_element_type=jnp.float32) + b1_ref[...][:, None]
    m = jnp.max(logits, axis=0, keepdims=True)
    lse = m + jnp.log(jnp.sum(jnp.exp(logits - m), axis=0, keepdims=True))
    out_ref[...] = logits - lse


def _head(parts, g0cols, params):
    full = lambda shape: pl.BlockSpec(shape, lambda i: tuple(0 for _ in shape))
    pspec = pl.BlockSpec((NC, 1, CH), lambda i: (0, 0, i))
    return pl.pallas_call(
        _head_body,
        grid=(CGRID,),
        in_specs=[pspec] * HID + [
            pl.BlockSpec((HID, 1, CH), lambda i: (0, 0, i)),
            full((HID,)), full((HID,)), full((HID,)),
            full((NCLS, HID)), full((NCLS,))],
        out_specs=pl.BlockSpec((NCLS, CH), lambda i: (0, i)),
        out_shape=jax.ShapeDtypeStruct((NCLS, NPAD), jnp.float32),
    )(*parts, g0cols, params['tag_b'], params['ln1_g'], params['ln1_b'],
      params['W1'].T, params['b1'])


def kernel(x, edge_index, edge_weight, categories_value, params):
    src = edge_index[0]
    dst = edge_index[1]
    emb_cat = params['emb_tables'].reshape(NCAT * N, EMB)

    DEBUG_JNP_PRE = False
    if DEBUG_JNP_PRE:
        deg_j = jnp.zeros((N,), jnp.float32).at[dst].add(edge_weight)
        degp = jnp.zeros((NC, 1, NPAD), jnp.float32).at[0, 0, :N].set(deg_j)
        idr = jnp.take(params['id_table'], categories_value[:, 0], axis=0)
        er = jnp.stack([
            jnp.take(params['emb_tables'][i], categories_value[:, i + 1],
                     axis=0) for i in range(NCAT)])
    else:
        degp, idr, er = _sc_pre(dst, edge_weight,
                                categories_value.reshape(N * 4),
                                params['id_table'], emb_cat)

    g0, g1, g2, g3, dinv2 = _encoder(
        x, idr, er[0], er[1], er[2],
        degp[0, 0, :N][:, None], degp[1, 0, :N][:, None], params)
    dinv = dinv2[:, 0]

    DEBUG_JNP_NORM = False
    if DEBUG_JNP_NORM:
        norm = dinv[src] * edge_weight * dinv[dst]
    else:
        norm = _sc_norm(src, dst, edge_weight, dinv)

    def cols(g):
        return jnp.pad(g.T, ((0, 0), (0, NPAD - N)))[:, None, :]

    g0c, g1c, g2c, g3c = cols(g0), cols(g1), cols(g2), cols(g3)
    tlist = [g3c[j, 0] for j in range(HID)]
    for gc in (g2c, g1c):
        parts = _sc_hop(*tlist, src, dst, norm)
        tlist = _combine(parts, gc)
    parts = _sc_hop(*tlist, src, dst, norm)
    out2 = _head(parts, g0c, params)
    return out2[:, :N].T


# speedup vs baseline: 4.0760x; 1.1430x over previous
import functools

import jax
import jax.numpy as jnp
from jax import lax
from jax.experimental import pallas as pl
from jax.experimental.pallas import tpu as pltpu
from jax.experimental.pallas import tpu_sc as plsc

N = 50000
E = 800000
F_NUM = 16
HID = 32
IDE = 16
EMB = 8
NCAT = 3
K = 3
NCLS = 2
EPS = 1e-5

NC = 2
NS = 16
L = 16

CH = 128
EC = E // NC
NCHUNK_C = EC // CH
ROWS_T = N // NS

BN = 400
GRID = N // BN

_mesh = plsc.VectorSubcoreMesh(core_axis_name="c", subcore_axis_name="s")


def _i16():
    return jax.lax.iota(jnp.int32, 16)


DEGC = 128
NDEGF = N // DEGC
DEGR = N - NDEGF * DEGC
NPAD = (NDEGF + 1) * DEGC
NPC = NPAD // DEGC
EMBC = 128
NEMBC = (N + EMBC - 1) // EMBC
TAIL_OFF = N - EMBC


def _sc_pre_body(dst_hbm, w_hbm, cv_hbm, idt_hbm, emb_hbm,
                 degp_hbm, idr_hbm, er_hbm,
                 deg_sh, zb_v, dstd_v, wd_v, cv_v, idx_v, idrows_v, erows_v,
                 gsem, osem):
    c = lax.axis_index("c")
    s = lax.axis_index("s")
    w = c * NS + s

    @pl.loop(0, DEGC // L)
    def _z(i):
        zb_v[pl.ds(i * L, L)] = jnp.zeros((L,), jnp.float32)

    nzc = jnp.where(s < NPC % NS, NPC // NS + 1, NPC // NS)

    @pl.loop(0, nzc)
    def _zd(k):
        cid = s + NS * k
        pltpu.sync_copy(zb_v, deg_sh.at[pl.ds(cid * DEGC, DEGC)])

    plsc.subcore_barrier()

    nec = jnp.where(s < 5, NCHUNK_C // NS + 1, NCHUNK_C // NS)

    @pl.loop(0, nec)
    def _e(k):
        cid = s + NS * k
        off = c * EC + cid * CH
        pltpu.sync_copy(dst_hbm.at[pl.ds(off, CH)], dstd_v.at[0])
        pltpu.sync_copy(w_hbm.at[pl.ds(off, CH)], wd_v)
        pltpu.sync_copy(wd_v, deg_sh.at[dstd_v.at[0]], add=True)

    plsc.subcore_barrier()

    @pl.loop(0, nzc)
    def _wd(k):
        cid = s + NS * k
        pltpu.sync_copy(deg_sh.at[pl.ds(cid * DEGC, DEGC)],
                        degp_hbm.at[c, 0, pl.ds(cid * DEGC, DEGC)])

    ngc = jnp.where(w < NEMBC % (NC * NS),
                    NEMBC // (NC * NS) + 1, NEMBC // (NC * NS))

    @pl.loop(0, ngc)
    def _g(k):
        cid = w + NC * NS * k
        off = jnp.where(cid == NEMBC - 1, TAIL_OFF, cid * EMBC)
        off = pl.multiple_of(off, 8)
        pltpu.sync_copy(cv_hbm.at[pl.ds(off * 4, EMBC * 4)], cv_v)
        for t in range(4):
            for i in range(EMBC // L):
                flat = (jnp.full((L,), i * L, jnp.int32) + _i16()) * 4 + t
                v = plsc.load_gather(cv_v, [flat])
                if t > 0:
                    v = v + jnp.full((L,), (t - 1) * N, jnp.int32)
                idx_v[t, pl.ds(i * L, L)] = v
        cp1 = pltpu.async_copy(idt_hbm.at[idx_v.at[0]], idrows_v, gsem)
        cp2 = pltpu.async_copy(emb_hbm.at[idx_v.at[1]], erows_v.at[0], gsem)
        cp3 = pltpu.async_copy(emb_hbm.at[idx_v.at[2]], erows_v.at[1], gsem)
        cp4 = pltpu.async_copy(emb_hbm.at[idx_v.at[3]], erows_v.at[2], gsem)
        cp1.wait()
        cp2.wait()
        cp3.wait()
        cp4.wait()
        o1 = pltpu.async_copy(idrows_v, idr_hbm.at[pl.ds(off, EMBC)], osem)
        o2 = pltpu.async_copy(erows_v.at[0], er_hbm.at[0, pl.ds(off, EMBC)],
                              osem)
        o3 = pltpu.async_copy(erows_v.at[1], er_hbm.at[1, pl.ds(off, EMBC)],
                              osem)
        o4 = pltpu.async_copy(erows_v.at[2], er_hbm.at[2, pl.ds(off, EMBC)],
                              osem)
        o1.wait()
        o2.wait()
        o3.wait()
        o4.wait()


@functools.partial(
    pl.kernel,
    out_type=[
        jax.ShapeDtypeStruct((NC, 1, NPAD), jnp.float32),
        jax.ShapeDtypeStruct((N, IDE), jnp.float32),
        jax.ShapeDtypeStruct((NCAT, N, EMB), jnp.float32),
    ],
    mesh=_mesh,
    scratch_types=[
        pltpu.VMEM_SHARED((NPAD,), jnp.float32),
        pltpu.VMEM((DEGC,), jnp.float32),
        pltpu.VMEM((1, CH), jnp.int32),
        pltpu.VMEM((CH,), jnp.float32),
        pltpu.VMEM((EMBC * 4,), jnp.int32),
        pltpu.VMEM((4, EMBC), jnp.int32),
        pltpu.VMEM((EMBC, IDE), jnp.float32),
        pltpu.VMEM((NCAT, EMBC, EMB), jnp.float32),
        pltpu.SemaphoreType.DMA,
        pltpu.SemaphoreType.DMA,
    ],
    compiler_params=pltpu.CompilerParams(needs_layout_passes=False, use_tc_tiling_on_sc=False),
)
def _sc_pre(dst_hbm, w_hbm, cv_hbm, idt_hbm, emb_hbm, *rest):
    _sc_pre_body(dst_hbm, w_hbm, cv_hbm, idt_hbm, emb_hbm, *rest)


def _sc_norm_body(src_hbm, dst_hbm, w_hbm, dinv_hbm, norm_hbm,
                  sidx_v, didx_v, wn_v, ds_v, dd_v, nrm_v, gsem):
    c = lax.axis_index("c")
    s = lax.axis_index("s")
    nec = jnp.where(s < NCHUNK_C % NS, NCHUNK_C // NS + 1, NCHUNK_C // NS)

    @pl.loop(0, nec)
    def _e(k):
        cid = s + NS * k
        off = c * EC + cid * CH
        pltpu.sync_copy(src_hbm.at[pl.ds(off, CH)], sidx_v.at[0])
        pltpu.sync_copy(dst_hbm.at[pl.ds(off, CH)], didx_v.at[0])
        pltpu.sync_copy(w_hbm.at[pl.ds(off, CH)], wn_v)
        c1 = pltpu.async_copy(dinv_hbm.at[sidx_v.at[0]], ds_v, gsem)
        c2 = pltpu.async_copy(dinv_hbm.at[didx_v.at[0]], dd_v, gsem)
        c1.wait()
        c2.wait()
        for i in range(CH // L):
            sl = pl.ds(i * L, L)
            nrm_v[sl] = ds_v[sl] * wn_v[sl] * dd_v[sl]
        pltpu.sync_copy(nrm_v, norm_hbm.at[pl.ds(off, CH)])


@functools.partial(
    pl.kernel,
    out_type=jax.ShapeDtypeStruct((E,), jnp.float32),
    mesh=_mesh,
    scratch_types=[
        pltpu.VMEM((1, CH), jnp.int32),
        pltpu.VMEM((1, CH), jnp.int32),
        pltpu.VMEM((CH,), jnp.float32),
        pltpu.VMEM((CH,), jnp.float32),
        pltpu.VMEM((CH,), jnp.float32),
        pltpu.VMEM((CH,), jnp.float32),
        pltpu.SemaphoreType.DMA,
    ],
    compiler_params=pltpu.CompilerParams(needs_layout_passes=False, use_tc_tiling_on_sc=False),
)
def _sc_norm(src_hbm, dst_hbm, w_hbm, dinv_hbm, *rest):
    _sc_norm_body(src_hbm, dst_hbm, w_hbm, dinv_hbm, *rest)


ZB = 4096
CHB = 640
NCHB = EC // CHB
GB = CHB // L


def _sc_hop_body(*args):
    tcols = args[:HID]
    src_hbm, dstl_hbm, norm_hbm = args[HID:HID + 3]
    parts = args[HID + 3:2 * HID + 3]
    accs = args[2 * HID + 3:3 * HID + 3]
    src_v, dst_v, norm_v, col_v, gsem, osem = args[3 * HID + 3:]
    c = lax.axis_index("c")
    s = lax.axis_index("s")

    @pl.loop(0, ZB // L)
    def _z(i):
        col_v[pl.ds(i * L, L)] = jnp.zeros((L,), jnp.float32)

    ZT = NPAD - (NPAD // ZB) * ZB

    def _zero_plane(acc):
        @pl.loop(0, NPAD // ZB)
        def _zp(i):
            pltpu.sync_copy(col_v.at[pl.ds(0, ZB)],
                            acc.at[pl.ds(i * ZB, ZB)])
        pltpu.sync_copy(col_v.at[pl.ds(0, ZT)],
                        acc.at[pl.ds((NPAD // ZB) * ZB, ZT)])

    for j in range(HID):
        @pl.when(s == j // 2)
        def _dz(acc=accs[j]):
            _zero_plane(acc)

    plsc.subcore_barrier()

    nec = jnp.where(s < NCHB % NS, NCHB // NS + 1, NCHB // NS)

    @pl.loop(0, nec)
    def _e(k):
        cid = s + NS * k
        off = c * EC + cid * CHB
        pltpu.sync_copy(src_hbm.at[pl.ds(off, CHB)], src_v.at[0])
        pltpu.sync_copy(dstl_hbm.at[pl.ds(off, CHB)], dst_v.at[0])
        pltpu.sync_copy(norm_hbm.at[pl.ds(off, CHB)], norm_v)
        gps = [pltpu.async_copy(tcols[j].at[src_v.at[0]],
                                col_v.at[pl.ds(j * CHB, CHB)], gsem)
               for j in range(HID)]
        for gp in gps:
            gp.wait()
        for g in range(GB):
            sl = pl.ds(g * L, L)
            nv = norm_v[sl]
            for j in range(HID):
                sl2 = pl.ds(j * CHB + g * L, L)
                col_v[sl2] = col_v[sl2] * nv
        cps = [pltpu.async_copy(col_v.at[pl.ds(j * CHB, CHB)],
                                accs[j].at[dst_v.at[0]], osem, add=True)
               for j in range(HID)]
        for cp in cps:
            cp.wait()

    plsc.subcore_barrier()

    for j in range(HID):
        @pl.when(s == j // 2)
        def _dw(acc=accs[j], part=parts[j]):
            pltpu.sync_copy(acc, part.at[c, 0])


@functools.partial(
    pl.kernel,
    out_type=[jax.ShapeDtypeStruct((NC, 1, NPAD), jnp.float32)
              for _ in range(HID)],
    mesh=_mesh,
    scratch_types=(
        [pltpu.VMEM_SHARED((NPAD,), jnp.float32) for _ in range(HID)] + [
            pltpu.VMEM((1, CHB), jnp.int32),
            pltpu.VMEM((1, CHB), jnp.int32),
            pltpu.VMEM((CHB,), jnp.float32),
            pltpu.VMEM((HID * CHB,), jnp.float32),
            pltpu.SemaphoreType.DMA,
            pltpu.SemaphoreType.DMA,
        ]),
    compiler_params=pltpu.CompilerParams(needs_layout_passes=False, use_tc_tiling_on_sc=False),
)
def _sc_hop(*args):
    _sc_hop_body(*args)


def _elu(x):
    return jnp.where(x > 0, x, jnp.exp(jnp.minimum(x, 0.0)) - 1.0)


def _enc_body(x_ref, idr_ref, e0_ref, e1_ref, e2_ref, dega_ref, degb_ref,
              w0_ref, b0_ref, wid_ref, bid_ref, wemb_ref, bemb_ref,
              lng_ref, lnb_ref, tw_ref,
              g0_ref, g1_ref, g2_ref, g3_ref, dinv_ref):
    h0 = _elu(jnp.dot(x_ref[...], w0_ref[...],
                      preferred_element_type=jnp.float32) + b0_ref[...])
    ide = _elu(jnp.dot(idr_ref[...], wid_ref[...],
                       preferred_element_type=jnp.float32) + bid_ref[...])
    cat = jnp.concatenate([e0_ref[...], e1_ref[...], e2_ref[...]], axis=1)
    ee = _elu(jnp.dot(cat, wemb_ref[...],
                      preferred_element_type=jnp.float32) + bemb_ref[...])
    h = jnp.concatenate([ide, h0, ee], axis=1)
    mu = jnp.mean(h, axis=-1, keepdims=True)
    var = jnp.mean((h - mu) ** 2, axis=-1, keepdims=True)
    hn = (h - mu) * jax.lax.rsqrt(var + EPS) * lng_ref[...] + lnb_ref[...]
    tw = tw_ref[...]
    g0_ref[...] = jnp.dot(hn, tw[0], preferred_element_type=jnp.float32)
    g1_ref[...] = jnp.dot(hn, tw[1], preferred_element_type=jnp.float32)
    g2_ref[...] = jnp.dot(hn, tw[2], preferred_element_type=jnp.float32)
    g3_ref[...] = jnp.dot(hn, tw[3], preferred_element_type=jnp.float32)
    deg = dega_ref[...] + degb_ref[...]
    dinv_ref[...] = jnp.where(
        deg > 0, jax.lax.rsqrt(jnp.maximum(deg, 1e-30)), 0.0)


def _encoder(x, idr, e0, e1, e2, dega, degb, params):
    D = IDE + HID + EMB * NCAT
    full = lambda shape: pl.BlockSpec(shape, lambda i: tuple(0 for _ in shape))
    row = lambda w: pl.BlockSpec((BN, w), lambda i: (i, 0))
    out_shapes = [jax.ShapeDtypeStruct((N, HID), jnp.float32) for _ in range(4)]
    out_shapes.append(jax.ShapeDtypeStruct((N, 1), jnp.float32))
    out_specs = [row(HID) for _ in range(4)] + [row(1)]
    return pl.pallas_call(
        _enc_body,
        grid=(GRID,),
        in_specs=[row(F_NUM), row(IDE), row(EMB), row(EMB), row(EMB),
                  row(1), row(1),
                  full((F_NUM, HID)), full((HID,)),
                  full((IDE, IDE)), full((IDE,)),
                  full((EMB * NCAT, EMB * NCAT)), full((EMB * NCAT,)),
                  full((D,)), full((D,)),
                  full((K + 1, D, HID))],
        out_specs=out_specs,
        out_shape=out_shapes,
    )(x, idr, e0, e1, e2, dega, degb,
      params['W0'], params['b0'], params['W_id'], params['b_id'],
      params['W_emb'], params['b_emb'], params['ln0_g'], params['ln0_b'],
      params['tag_W'])


CGRID = NPAD // CH


def _comb_body(*refs):
    p_refs = refs[:HID]
    gc_ref = refs[HID]
    out_refs = refs[HID + 1:]
    for j in range(HID):
        out_refs[j][...] = (p_refs[j][0, 0] + p_refs[j][1, 0]
                            + gc_ref[j, 0, :])


def _combine(parts, gcols):
    pspec = pl.BlockSpec((NC, 1, CH), lambda i: (0, 0, i))
    return pl.pallas_call(
        _comb_body,
        grid=(CGRID,),
        in_specs=[pspec] * HID + [pl.BlockSpec((HID, 1, CH),
                                               lambda i: (0, 0, i))],
        out_specs=[pl.BlockSpec((CH,), lambda i: (i,))] * HID,
        out_shape=[jax.ShapeDtypeStruct((NPAD,), jnp.float32)] * HID,
    )(*parts, gcols)


def _head_body(*refs):
    p_refs = refs[:HID]
    (gc_ref, tb_ref, lng_ref, lnb_ref, w1t_ref, b1_ref) = refs[HID:HID + 6]
    out_ref = refs[HID + 6]
    t = jnp.concatenate(
        [p_refs[j][...].sum(axis=0) for j in range(HID)], axis=0
    ) + gc_ref[:, 0, :]
    t = jnp.maximum(t + tb_ref[...][:, None], 0.0)
    mu = jnp.mean(t, axis=0, keepdims=True)
    var = jnp.mean((t - mu) ** 2, axis=0, keepdims=True)
    tn = ((t - mu) * jax.lax.rsqrt(var + EPS) * lng_ref[...][:, None]
          + lnb_ref[...][:, None])
    logits = jnp.dot(w1t_ref[...], tn,
                     preferred_element_type=jnp.float32) + b1_ref[...][:, None]
    m = jnp.max(logits, axis=0, keepdims=True)
    lse = m + jnp.log(jnp.sum(jnp.exp(logits - m), axis=0, keepdims=True))
    out_ref[...] = logits - lse


def _head(parts, g0cols, params):
    full = lambda shape: pl.BlockSpec(shape, lambda i: tuple(0 for _ in shape))
    pspec = pl.BlockSpec((NC, 1, CH), lambda i: (0, 0, i))
    return pl.pallas_call(
        _head_body,
        grid=(CGRID,),
        in_specs=[pspec] * HID + [
            pl.BlockSpec((HID, 1, CH), lambda i: (0, 0, i)),
            full((HID,)), full((HID,)), full((HID,)),
            full((NCLS, HID)), full((NCLS,))],
        out_specs=pl.BlockSpec((NCLS, CH), lambda i: (0, i)),
        out_shape=jax.ShapeDtypeStruct((NCLS, NPAD), jnp.float32),
    )(*parts, g0cols, params['tag_b'], params['ln1_g'], params['ln1_b'],
      params['W1'].T, params['b1'])


def kernel(x, edge_index, edge_weight, categories_value, params):
    src = edge_index[0]
    dst = edge_index[1]
    emb_cat = params['emb_tables'].reshape(NCAT * N, EMB)

    DEBUG_JNP_PRE = False
    if DEBUG_JNP_PRE:
        deg_j = jnp.zeros((N,), jnp.float32).at[dst].add(edge_weight)
        degp = jnp.zeros((NC, 1, NPAD), jnp.float32).at[0, 0, :N].set(deg_j)
        idr = jnp.take(params['id_table'], categories_value[:, 0], axis=0)
        er = jnp.stack([
            jnp.take(params['emb_tables'][i], categories_value[:, i + 1],
                     axis=0) for i in range(NCAT)])
    else:
        degp, idr, er = _sc_pre(dst, edge_weight,
                                categories_value.reshape(N * 4),
                                params['id_table'], emb_cat)

    g0, g1, g2, g3, dinv2 = _encoder(
        x, idr, er[0], er[1], er[2],
        degp[0, 0, :N][:, None], degp[1, 0, :N][:, None], params)
    dinv = dinv2[:, 0]

    DEBUG_JNP_NORM = False
    if DEBUG_JNP_NORM:
        norm = dinv[src] * edge_weight * dinv[dst]
    else:
        norm = _sc_norm(src, dst, edge_weight, dinv)

    def cols(g):
        return jnp.pad(g.T, ((0, 0), (0, NPAD - N)))[:, None, :]

    g0c, g1c, g2c, g3c = cols(g0), cols(g1), cols(g2), cols(g3)
    tlist = [g3c[j, 0] for j in range(HID)]
    for gc in (g2c, g1c):
        parts = _sc_hop(*tlist, src, dst, norm)
        tlist = _combine(parts, gc)
    parts = _sc_hop(*tlist, src, dst, norm)
    out2 = _head(parts, g0c, params)
    return out2[:, :N].T


# speedup vs baseline: 4.2368x; 1.0395x over previous
import functools

import jax
import jax.numpy as jnp
from jax import lax
from jax.experimental import pallas as pl
from jax.experimental.pallas import tpu as pltpu
from jax.experimental.pallas import tpu_sc as plsc

N = 50000
E = 800000
F_NUM = 16
HID = 32
IDE = 16
EMB = 8
NCAT = 3
K = 3
NCLS = 2
EPS = 1e-5

NC = 2
NS = 16
L = 16

CH = 128
EC = E // NC
NCHUNK_C = EC // CH
ROWS_T = N // NS

BN = 400
GRID = N // BN

_mesh = plsc.VectorSubcoreMesh(core_axis_name="c", subcore_axis_name="s")


def _i16():
    return jax.lax.iota(jnp.int32, 16)


DEGC = 128
NDEGF = N // DEGC
DEGR = N - NDEGF * DEGC
NPAD = (NDEGF + 1) * DEGC
NPC = NPAD // DEGC
EMBC = 128
NEMBC = (N + EMBC - 1) // EMBC
TAIL_OFF = N - EMBC


def _sc_pre_body(dst_hbm, w_hbm, cv_hbm, idt_hbm, emb_hbm,
                 degp_hbm, idr_hbm, er_hbm,
                 deg_sh, zb_v, dstd_v, wd_v, cv_v, idx_v, idrows_v, erows_v,
                 gsem, osem):
    c = lax.axis_index("c")
    s = lax.axis_index("s")
    w = c * NS + s

    @pl.loop(0, DEGC // L)
    def _z(i):
        zb_v[pl.ds(i * L, L)] = jnp.zeros((L,), jnp.float32)

    nzc = jnp.where(s < NPC % NS, NPC // NS + 1, NPC // NS)

    @pl.loop(0, nzc)
    def _zd(k):
        cid = s + NS * k
        pltpu.sync_copy(zb_v, deg_sh.at[pl.ds(cid * DEGC, DEGC)])

    plsc.subcore_barrier()

    nec = jnp.where(s < 5, NCHUNK_C // NS + 1, NCHUNK_C // NS)

    @pl.loop(0, nec)
    def _e(k):
        cid = s + NS * k
        off = c * EC + cid * CH
        pltpu.sync_copy(dst_hbm.at[pl.ds(off, CH)], dstd_v.at[0])
        pltpu.sync_copy(w_hbm.at[pl.ds(off, CH)], wd_v)
        pltpu.sync_copy(wd_v, deg_sh.at[dstd_v.at[0]], add=True)

    plsc.subcore_barrier()

    @pl.loop(0, nzc)
    def _wd(k):
        cid = s + NS * k
        pltpu.sync_copy(deg_sh.at[pl.ds(cid * DEGC, DEGC)],
                        degp_hbm.at[c, 0, pl.ds(cid * DEGC, DEGC)])

    ngc = jnp.where(w < NEMBC % (NC * NS),
                    NEMBC // (NC * NS) + 1, NEMBC // (NC * NS))

    @pl.loop(0, ngc)
    def _g(k):
        cid = w + NC * NS * k
        off = jnp.where(cid == NEMBC - 1, TAIL_OFF, cid * EMBC)
        off = pl.multiple_of(off, 8)
        pltpu.sync_copy(cv_hbm.at[pl.ds(off * 4, EMBC * 4)], cv_v)
        for t in range(4):
            for i in range(EMBC // L):
                flat = (jnp.full((L,), i * L, jnp.int32) + _i16()) * 4 + t
                v = plsc.load_gather(cv_v, [flat])
                if t > 0:
                    v = v + jnp.full((L,), (t - 1) * N, jnp.int32)
                idx_v[t, pl.ds(i * L, L)] = v
        cp1 = pltpu.async_copy(idt_hbm.at[idx_v.at[0]], idrows_v, gsem)
        cp2 = pltpu.async_copy(emb_hbm.at[idx_v.at[1]], erows_v.at[0], gsem)
        cp3 = pltpu.async_copy(emb_hbm.at[idx_v.at[2]], erows_v.at[1], gsem)
        cp4 = pltpu.async_copy(emb_hbm.at[idx_v.at[3]], erows_v.at[2], gsem)
        cp1.wait()
        cp2.wait()
        cp3.wait()
        cp4.wait()
        o1 = pltpu.async_copy(idrows_v, idr_hbm.at[pl.ds(off, EMBC)], osem)
        o2 = pltpu.async_copy(erows_v.at[0], er_hbm.at[0, pl.ds(off, EMBC)],
                              osem)
        o3 = pltpu.async_copy(erows_v.at[1], er_hbm.at[1, pl.ds(off, EMBC)],
                              osem)
        o4 = pltpu.async_copy(erows_v.at[2], er_hbm.at[2, pl.ds(off, EMBC)],
                              osem)
        o1.wait()
        o2.wait()
        o3.wait()
        o4.wait()


@functools.partial(
    pl.kernel,
    out_type=[
        jax.ShapeDtypeStruct((NC, 1, NPAD), jnp.float32),
        jax.ShapeDtypeStruct((N, IDE), jnp.float32),
        jax.ShapeDtypeStruct((NCAT, N, EMB), jnp.float32),
    ],
    mesh=_mesh,
    scratch_types=[
        pltpu.VMEM_SHARED((NPAD,), jnp.float32),
        pltpu.VMEM((DEGC,), jnp.float32),
        pltpu.VMEM((1, CH), jnp.int32),
        pltpu.VMEM((CH,), jnp.float32),
        pltpu.VMEM((EMBC * 4,), jnp.int32),
        pltpu.VMEM((4, EMBC), jnp.int32),
        pltpu.VMEM((EMBC, IDE), jnp.float32),
        pltpu.VMEM((NCAT, EMBC, EMB), jnp.float32),
        pltpu.SemaphoreType.DMA,
        pltpu.SemaphoreType.DMA,
    ],
    compiler_params=pltpu.CompilerParams(needs_layout_passes=False, use_tc_tiling_on_sc=False),
)
def _sc_pre(dst_hbm, w_hbm, cv_hbm, idt_hbm, emb_hbm, *rest):
    _sc_pre_body(dst_hbm, w_hbm, cv_hbm, idt_hbm, emb_hbm, *rest)


def _sc_norm_body(src_hbm, dst_hbm, w_hbm, dinv_hbm, norm_hbm,
                  sidx_v, didx_v, wn_v, ds_v, dd_v, nrm_v, gsem):
    c = lax.axis_index("c")
    s = lax.axis_index("s")
    nec = jnp.where(s < NCHUNK_C % NS, NCHUNK_C // NS + 1, NCHUNK_C // NS)

    @pl.loop(0, nec)
    def _e(k):
        cid = s + NS * k
        off = c * EC + cid * CH
        pltpu.sync_copy(src_hbm.at[pl.ds(off, CH)], sidx_v.at[0])
        pltpu.sync_copy(dst_hbm.at[pl.ds(off, CH)], didx_v.at[0])
        pltpu.sync_copy(w_hbm.at[pl.ds(off, CH)], wn_v)
        c1 = pltpu.async_copy(dinv_hbm.at[sidx_v.at[0]], ds_v, gsem)
        c2 = pltpu.async_copy(dinv_hbm.at[didx_v.at[0]], dd_v, gsem)
        c1.wait()
        c2.wait()
        for i in range(CH // L):
            sl = pl.ds(i * L, L)
            nrm_v[sl] = ds_v[sl] * wn_v[sl] * dd_v[sl]
        pltpu.sync_copy(nrm_v, norm_hbm.at[pl.ds(off, CH)])


@functools.partial(
    pl.kernel,
    out_type=jax.ShapeDtypeStruct((E,), jnp.float32),
    mesh=_mesh,
    scratch_types=[
        pltpu.VMEM((1, CH), jnp.int32),
        pltpu.VMEM((1, CH), jnp.int32),
        pltpu.VMEM((CH,), jnp.float32),
        pltpu.VMEM((CH,), jnp.float32),
        pltpu.VMEM((CH,), jnp.float32),
        pltpu.VMEM((CH,), jnp.float32),
        pltpu.SemaphoreType.DMA,
    ],
    compiler_params=pltpu.CompilerParams(needs_layout_passes=False, use_tc_tiling_on_sc=False),
)
def _sc_norm(src_hbm, dst_hbm, w_hbm, dinv_hbm, *rest):
    _sc_norm_body(src_hbm, dst_hbm, w_hbm, dinv_hbm, *rest)


ZB = 4096
CHB = 640
NCHB = EC // CHB
GB = CHB // L


def _sc_hop_body(*args):
    tcols = args[:HID]
    src_hbm, dstl_hbm, norm_hbm = args[HID:HID + 3]
    parts = args[HID + 3:2 * HID + 3]
    accs = args[2 * HID + 3:3 * HID + 3]
    src_v, dst_v, norm_v, col_v, gsem, osem = args[3 * HID + 3:]
    c = lax.axis_index("c")
    s = lax.axis_index("s")

    @pl.loop(0, ZB // L)
    def _z(i):
        col_v[pl.ds(i * L, L)] = jnp.zeros((L,), jnp.float32)

    ZT = NPAD - (NPAD // ZB) * ZB

    def _zero_plane(acc):
        @pl.loop(0, NPAD // ZB)
        def _zp(i):
            pltpu.sync_copy(col_v.at[pl.ds(0, ZB)],
                            acc.at[pl.ds(i * ZB, ZB)])
        pltpu.sync_copy(col_v.at[pl.ds(0, ZT)],
                        acc.at[pl.ds((NPAD // ZB) * ZB, ZT)])

    for j in range(HID):
        @pl.when(s == j // 2)
        def _dz(acc=accs[j]):
            _zero_plane(acc)

    plsc.subcore_barrier()

    nec = jnp.where(s < NCHB % NS, NCHB // NS + 1, NCHB // NS)

    @pl.loop(0, nec)
    def _e(k):
        cid = s + NS * k
        off = c * EC + cid * CHB
        pltpu.sync_copy(src_hbm.at[pl.ds(off, CHB)], src_v.at[0])
        pltpu.sync_copy(dstl_hbm.at[pl.ds(off, CHB)], dst_v.at[0])
        pltpu.sync_copy(norm_hbm.at[pl.ds(off, CHB)], norm_v)
        gps = [pltpu.async_copy(tcols[j].at[src_v.at[0]],
                                col_v.at[pl.ds(j * CHB, CHB)], gsem)
               for j in range(HID)]
        cps = []
        for j in range(HID):
            gps[j].wait()
            for g in range(GB):
                sl2 = pl.ds(j * CHB + g * L, L)
                col_v[sl2] = col_v[sl2] * norm_v[pl.ds(g * L, L)]
            cps.append(pltpu.async_copy(col_v.at[pl.ds(j * CHB, CHB)],
                                        accs[j].at[dst_v.at[0]], osem,
                                        add=True))
        for cp in cps:
            cp.wait()

    plsc.subcore_barrier()

    for j in range(HID):
        @pl.when(s == j // 2)
        def _dw(acc=accs[j], part=parts[j]):
            pltpu.sync_copy(acc, part.at[c, 0])


@functools.partial(
    pl.kernel,
    out_type=[jax.ShapeDtypeStruct((NC, 1, NPAD), jnp.float32)
              for _ in range(HID)],
    mesh=_mesh,
    scratch_types=(
        [pltpu.VMEM_SHARED((NPAD,), jnp.float32) for _ in range(HID)] + [
            pltpu.VMEM((1, CHB), jnp.int32),
            pltpu.VMEM((1, CHB), jnp.int32),
            pltpu.VMEM((CHB,), jnp.float32),
            pltpu.VMEM((HID * CHB,), jnp.float32),
            pltpu.SemaphoreType.DMA,
            pltpu.SemaphoreType.DMA,
        ]),
    compiler_params=pltpu.CompilerParams(needs_layout_passes=False, use_tc_tiling_on_sc=False),
)
def _sc_hop(*args):
    _sc_hop_body(*args)


def _elu(x):
    return jnp.where(x > 0, x, jnp.exp(jnp.minimum(x, 0.0)) - 1.0)


def _enc_body(x_ref, idr_ref, e0_ref, e1_ref, e2_ref, dega_ref, degb_ref,
              w0_ref, b0_ref, wid_ref, bid_ref, wemb_ref, bemb_ref,
              lng_ref, lnb_ref, tw_ref,
              g0_ref, g1_ref, g2_ref, g3_ref, dinv_ref):
    h0 = _elu(jnp.dot(x_ref[...], w0_ref[...],
                      preferred_element_type=jnp.float32) + b0_ref[...])
    ide = _elu(jnp.dot(idr_ref[...], wid_ref[...],
                       preferred_element_type=jnp.float32) + bid_ref[...])
    cat = jnp.concatenate([e0_ref[...], e1_ref[...], e2_ref[...]], axis=1)
    ee = _elu(jnp.dot(cat, wemb_ref[...],
                      preferred_element_type=jnp.float32) + bemb_ref[...])
    h = jnp.concatenate([ide, h0, ee], axis=1)
    mu = jnp.mean(h, axis=-1, keepdims=True)
    var = jnp.mean((h - mu) ** 2, axis=-1, keepdims=True)
    hn = (h - mu) * jax.lax.rsqrt(var + EPS) * lng_ref[...] + lnb_ref[...]
    tw = tw_ref[...]
    g0_ref[...] = jnp.dot(hn, tw[0], preferred_element_type=jnp.float32)
    g1_ref[...] = jnp.dot(hn, tw[1], preferred_element_type=jnp.float32)
    g2_ref[...] = jnp.dot(hn, tw[2], preferred_element_type=jnp.float32)
    g3_ref[...] = jnp.dot(hn, tw[3], preferred_element_type=jnp.float32)
    deg = dega_ref[...] + degb_ref[...]
    dinv_ref[...] = jnp.where(
        deg > 0, jax.lax.rsqrt(jnp.maximum(deg, 1e-30)), 0.0)


def _encoder(x, idr, e0, e1, e2, dega, degb, params):
    D = IDE + HID + EMB * NCAT
    full = lambda shape: pl.BlockSpec(shape, lambda i: tuple(0 for _ in shape))
    row = lambda w: pl.BlockSpec((BN, w), lambda i: (i, 0))
    out_shapes = [jax.ShapeDtypeStruct((N, HID), jnp.float32) for _ in range(4)]
    out_shapes.append(jax.ShapeDtypeStruct((N, 1), jnp.float32))
    out_specs = [row(HID) for _ in range(4)] + [row(1)]
    return pl.pallas_call(
        _enc_body,
        grid=(GRID,),
        in_specs=[row(F_NUM), row(IDE), row(EMB), row(EMB), row(EMB),
                  row(1), row(1),
                  full((F_NUM, HID)), full((HID,)),
                  full((IDE, IDE)), full((IDE,)),
                  full((EMB * NCAT, EMB * NCAT)), full((EMB * NCAT,)),
                  full((D,)), full((D,)),
                  full((K + 1, D, HID))],
        out_specs=out_specs,
        out_shape=out_shapes,
    )(x, idr, e0, e1, e2, dega, degb,
      params['W0'], params['b0'], params['W_id'], params['b_id'],
      params['W_emb'], params['b_emb'], params['ln0_g'], params['ln0_b'],
      params['tag_W'])


CGRID = NPAD // CH


def _comb_body(*refs):
    p_refs = refs[:HID]
    gc_ref = refs[HID]
    out_refs = refs[HID + 1:]
    for j in range(HID):
        out_refs[j][...] = (p_refs[j][0, 0] + p_refs[j][1, 0]
                            + gc_ref[j, 0, :])


def _combine(parts, gcols):
    pspec = pl.BlockSpec((NC, 1, CH), lambda i: (0, 0, i))
    return pl.pallas_call(
        _comb_body,
        grid=(CGRID,),
        in_specs=[pspec] * HID + [pl.BlockSpec((HID, 1, CH),
                                               lambda i: (0, 0, i))],
        out_specs=[pl.BlockSpec((CH,), lambda i: (i,))] * HID,
        out_shape=[jax.ShapeDtypeStruct((NPAD,), jnp.float32)] * HID,
    )(*parts, gcols)


def _head_body(*refs):
    p_refs = refs[:HID]
    (gc_ref, tb_ref, lng_ref, lnb_ref, w1t_ref, b1_ref) = refs[HID:HID + 6]
    out_ref = refs[HID + 6]
    t = jnp.concatenate(
        [p_refs[j][...].sum(axis=0) for j in range(HID)], axis=0
    ) + gc_ref[:, 0, :]
    t = jnp.maximum(t + tb_ref[...][:, None], 0.0)
    mu = jnp.mean(t, axis=0, keepdims=True)
    var = jnp.mean((t - mu) ** 2, axis=0, keepdims=True)
    tn = ((t - mu) * jax.lax.rsqrt(var + EPS) * lng_ref[...][:, None]
          + lnb_ref[...][:, None])
    logits = jnp.dot(w1t_ref[...], tn,
                     preferred_element_type=jnp.float32) + b1_ref[...][:, None]
    m = jnp.max(logits, axis=0, keepdims=True)
    lse = m + jnp.log(jnp.sum(jnp.exp(logits - m), axis=0, keepdims=True))
    out_ref[...] = logits - lse


def _head(parts, g0cols, params):
    full = lambda shape: pl.BlockSpec(shape, lambda i: tuple(0 for _ in shape))
    pspec = pl.BlockSpec((NC, 1, CH), lambda i: (0, 0, i))
    return pl.pallas_call(
        _head_body,
        grid=(CGRID,),
        in_specs=[pspec] * HID + [
            pl.BlockSpec((HID, 1, CH), lambda i: (0, 0, i)),
            full((HID,)), full((HID,)), full((HID,)),
            full((NCLS, HID)), full((NCLS,))],
        out_specs=pl.BlockSpec((NCLS, CH), lambda i: (0, i)),
        out_shape=jax.ShapeDtypeStruct((NCLS, NPAD), jnp.float32),
    )(*parts, g0cols, params['tag_b'], params['ln1_g'], params['ln1_b'],
      params['W1'].T, params['b1'])


def kernel(x, edge_index, edge_weight, categories_value, params):
    src = edge_index[0]
    dst = edge_index[1]
    emb_cat = params['emb_tables'].reshape(NCAT * N, EMB)

    DEBUG_JNP_PRE = False
    if DEBUG_JNP_PRE:
        deg_j = jnp.zeros((N,), jnp.float32).at[dst].add(edge_weight)
        degp = jnp.zeros((NC, 1, NPAD), jnp.float32).at[0, 0, :N].set(deg_j)
        idr = jnp.take(params['id_table'], categories_value[:, 0], axis=0)
        er = jnp.stack([
            jnp.take(params['emb_tables'][i], categories_value[:, i + 1],
                     axis=0) for i in range(NCAT)])
    else:
        degp, idr, er = _sc_pre(dst, edge_weight,
                                categories_value.reshape(N * 4),
                                params['id_table'], emb_cat)

    g0, g1, g2, g3, dinv2 = _encoder(
        x, idr, er[0], er[1], er[2],
        degp[0, 0, :N][:, None], degp[1, 0, :N][:, None], params)
    dinv = dinv2[:, 0]

    DEBUG_JNP_NORM = False
    if DEBUG_JNP_NORM:
        norm = dinv[src] * edge_weight * dinv[dst]
    else:
        norm = _sc_norm(src, dst, edge_weight, dinv)

    def cols(g):
        return jnp.pad(g.T, ((0, 0), (0, NPAD - N)))[:, None, :]

    g0c, g1c, g2c, g3c = cols(g0), cols(g1), cols(g2), cols(g3)
    tlist = [g3c[j, 0] for j in range(HID)]
    for gc in (g2c, g1c):
        parts = _sc_hop(*tlist, src, dst, norm)
        tlist = _combine(parts, gc)
    parts = _sc_hop(*tlist, src, dst, norm)
    out2 = _head(parts, g0c, params)
    return out2[:, :N].T


# speedup vs baseline: 4.2375x; 1.0002x over previous
import functools

import jax
import jax.numpy as jnp
from jax import lax
from jax.experimental import pallas as pl
from jax.experimental.pallas import tpu as pltpu
from jax.experimental.pallas import tpu_sc as plsc

N = 50000
E = 800000
F_NUM = 16
HID = 32
IDE = 16
EMB = 8
NCAT = 3
K = 3
NCLS = 2
EPS = 1e-5

NC = 2
NS = 16
L = 16

CH = 128
EC = E // NC
NCHUNK_C = EC // CH

BN = 400
GRID = N // BN

_mesh = plsc.VectorSubcoreMesh(core_axis_name="c", subcore_axis_name="s")


def _i16():
    return jax.lax.iota(jnp.int32, 16)


DEGC = 128
NDEGF = N // DEGC
DEGR = N - NDEGF * DEGC
NPAD = (NDEGF + 1) * DEGC
NPC = NPAD // DEGC
EMBC = 128
NEMBC = (N + EMBC - 1) // EMBC
TAIL_OFF = N - EMBC


def _sc_pre_body(dst_hbm, w_hbm, cv_hbm, idt_hbm, emb_hbm,
                 degp_hbm, idr_hbm, er_hbm,
                 deg_sh, zb_v, dstd_v, wd_v, cv_v, idx_v, idrows_v, erows_v,
                 gsem, osem):
    c = lax.axis_index("c")
    s = lax.axis_index("s")
    w = c * NS + s

    @pl.loop(0, DEGC // L)
    def _z(i):
        zb_v[pl.ds(i * L, L)] = jnp.zeros((L,), jnp.float32)

    nzc = jnp.where(s < NPC % NS, NPC // NS + 1, NPC // NS)

    @pl.loop(0, nzc)
    def _zd(k):
        cid = s + NS * k
        pltpu.sync_copy(zb_v, deg_sh.at[pl.ds(cid * DEGC, DEGC)])

    plsc.subcore_barrier()

    nec = jnp.where(s < 5, NCHUNK_C // NS + 1, NCHUNK_C // NS)

    @pl.loop(0, nec)
    def _e(k):
        cid = s + NS * k
        off = c * EC + cid * CH
        pltpu.sync_copy(dst_hbm.at[pl.ds(off, CH)], dstd_v.at[0])
        pltpu.sync_copy(w_hbm.at[pl.ds(off, CH)], wd_v)
        pltpu.sync_copy(wd_v, deg_sh.at[dstd_v.at[0]], add=True)

    plsc.subcore_barrier()

    @pl.loop(0, nzc)
    def _wd(k):
        cid = s + NS * k
        pltpu.sync_copy(deg_sh.at[pl.ds(cid * DEGC, DEGC)],
                        degp_hbm.at[c, 0, pl.ds(cid * DEGC, DEGC)])

    ngc = jnp.where(w < NEMBC % (NC * NS),
                    NEMBC // (NC * NS) + 1, NEMBC // (NC * NS))

    @pl.loop(0, ngc)
    def _g(k):
        cid = w + NC * NS * k
        off = jnp.where(cid == NEMBC - 1, TAIL_OFF, cid * EMBC)
        off = pl.multiple_of(off, 8)
        pltpu.sync_copy(cv_hbm.at[pl.ds(off * 4, EMBC * 4)], cv_v)
        for t in range(4):
            for i in range(EMBC // L):
                flat = (jnp.full((L,), i * L, jnp.int32) + _i16()) * 4 + t
                v = plsc.load_gather(cv_v, [flat])
                if t > 0:
                    v = v + jnp.full((L,), (t - 1) * N, jnp.int32)
                idx_v[t, pl.ds(i * L, L)] = v
        cp1 = pltpu.async_copy(idt_hbm.at[idx_v.at[0]], idrows_v, gsem)
        cp2 = pltpu.async_copy(emb_hbm.at[idx_v.at[1]], erows_v.at[0], gsem)
        cp3 = pltpu.async_copy(emb_hbm.at[idx_v.at[2]], erows_v.at[1], gsem)
        cp4 = pltpu.async_copy(emb_hbm.at[idx_v.at[3]], erows_v.at[2], gsem)
        cp1.wait()
        cp2.wait()
        cp3.wait()
        cp4.wait()
        o1 = pltpu.async_copy(idrows_v, idr_hbm.at[pl.ds(off, EMBC)], osem)
        o2 = pltpu.async_copy(erows_v.at[0], er_hbm.at[0, pl.ds(off, EMBC)],
                              osem)
        o3 = pltpu.async_copy(erows_v.at[1], er_hbm.at[1, pl.ds(off, EMBC)],
                              osem)
        o4 = pltpu.async_copy(erows_v.at[2], er_hbm.at[2, pl.ds(off, EMBC)],
                              osem)
        o1.wait()
        o2.wait()
        o3.wait()
        o4.wait()


@functools.partial(
    pl.kernel,
    out_type=[
        jax.ShapeDtypeStruct((NC, 1, NPAD), jnp.float32),
        jax.ShapeDtypeStruct((N, IDE), jnp.float32),
        jax.ShapeDtypeStruct((NCAT, N, EMB), jnp.float32),
    ],
    mesh=_mesh,
    scratch_types=[
        pltpu.VMEM_SHARED((NPAD,), jnp.float32),
        pltpu.VMEM((DEGC,), jnp.float32),
        pltpu.VMEM((1, CH), jnp.int32),
        pltpu.VMEM((CH,), jnp.float32),
        pltpu.VMEM((EMBC * 4,), jnp.int32),
        pltpu.VMEM((4, EMBC), jnp.int32),
        pltpu.VMEM((EMBC, IDE), jnp.float32),
        pltpu.VMEM((NCAT, EMBC, EMB), jnp.float32),
        pltpu.SemaphoreType.DMA,
        pltpu.SemaphoreType.DMA,
    ],
    compiler_params=pltpu.CompilerParams(needs_layout_passes=False, use_tc_tiling_on_sc=False),
)
def _sc_pre(dst_hbm, w_hbm, cv_hbm, idt_hbm, emb_hbm, *rest):
    _sc_pre_body(dst_hbm, w_hbm, cv_hbm, idt_hbm, emb_hbm, *rest)


def _sc_norm_body(src_hbm, dst_hbm, w_hbm, dinv_hbm, norm_hbm,
                  sidx_v, didx_v, wn_v, ds_v, dd_v, nrm_v, gsem):
    c = lax.axis_index("c")
    s = lax.axis_index("s")
    nec = jnp.where(s < NCHUNK_C % NS, NCHUNK_C // NS + 1, NCHUNK_C // NS)

    @pl.loop(0, nec)
    def _e(k):
        cid = s + NS * k
        off = c * EC + cid * CH
        pltpu.sync_copy(src_hbm.at[pl.ds(off, CH)], sidx_v.at[0])
        pltpu.sync_copy(dst_hbm.at[pl.ds(off, CH)], didx_v.at[0])
        pltpu.sync_copy(w_hbm.at[pl.ds(off, CH)], wn_v)
        c1 = pltpu.async_copy(dinv_hbm.at[sidx_v.at[0]], ds_v, gsem)
        c2 = pltpu.async_copy(dinv_hbm.at[didx_v.at[0]], dd_v, gsem)
        c1.wait()
        c2.wait()
        for i in range(CH // L):
            sl = pl.ds(i * L, L)
            nrm_v[sl] = ds_v[sl] * wn_v[sl] * dd_v[sl]
        pltpu.sync_copy(nrm_v, norm_hbm.at[pl.ds(off, CH)])


@functools.partial(
    pl.kernel,
    out_type=jax.ShapeDtypeStruct((E,), jnp.float32),
    mesh=_mesh,
    scratch_types=[
        pltpu.VMEM((1, CH), jnp.int32),
        pltpu.VMEM((1, CH), jnp.int32),
        pltpu.VMEM((CH,), jnp.float32),
        pltpu.VMEM((CH,), jnp.float32),
        pltpu.VMEM((CH,), jnp.float32),
        pltpu.VMEM((CH,), jnp.float32),
        pltpu.SemaphoreType.DMA,
    ],
    compiler_params=pltpu.CompilerParams(needs_layout_passes=False, use_tc_tiling_on_sc=False),
)
def _sc_norm(src_hbm, dst_hbm, w_hbm, dinv_hbm, *rest):
    _sc_norm_body(src_hbm, dst_hbm, w_hbm, dinv_hbm, *rest)


ZB = 4096
CHB = 640
NCHB = EC // CHB
GB = CHB // L


def _sc_hop_body(*args):
    tcols = args[:HID]
    src_hbm, dstl_hbm, norm_hbm = args[HID:HID + 3]
    parts = args[HID + 3:2 * HID + 3]
    accs = args[2 * HID + 3:3 * HID + 3]
    src_v, dst_v, norm_v, col_v, gsem, osem = args[3 * HID + 3:]
    c = lax.axis_index("c")
    s = lax.axis_index("s")

    @pl.loop(0, ZB // L)
    def _z(i):
        col_v[pl.ds(i * L, L)] = jnp.zeros((L,), jnp.float32)

    ZT = NPAD - (NPAD // ZB) * ZB

    def _zero_plane(acc):
        @pl.loop(0, NPAD // ZB)
        def _zp(i):
            pltpu.sync_copy(col_v.at[pl.ds(0, ZB)],
                            acc.at[pl.ds(i * ZB, ZB)])
        pltpu.sync_copy(col_v.at[pl.ds(0, ZT)],
                        acc.at[pl.ds((NPAD // ZB) * ZB, ZT)])

    for j in range(HID):
        @pl.when(s == j // 2)
        def _dz(acc=accs[j]):
            _zero_plane(acc)

    plsc.subcore_barrier()

    nec = jnp.where(s < NCHB % NS, NCHB // NS + 1, NCHB // NS)

    @pl.loop(0, nec)
    def _e(k):
        cid = s + NS * k
        off = c * EC + cid * CHB
        pltpu.sync_copy(src_hbm.at[pl.ds(off, CHB)], src_v.at[0])
        pltpu.sync_copy(dstl_hbm.at[pl.ds(off, CHB)], dst_v.at[0])
        pltpu.sync_copy(norm_hbm.at[pl.ds(off, CHB)], norm_v)
        gps = [pltpu.async_copy(tcols[j].at[src_v.at[0]],
                                col_v.at[pl.ds(j * CHB, CHB)], gsem)
               for j in range(HID)]
        cps = []
        for j in range(HID):
            gps[j].wait()
            for g in range(GB):
                sl2 = pl.ds(j * CHB + g * L, L)
                col_v[sl2] = col_v[sl2] * norm_v[pl.ds(g * L, L)]
            cps.append(pltpu.async_copy(col_v.at[pl.ds(j * CHB, CHB)],
                                        accs[j].at[dst_v.at[0]], osem,
                                        add=True))
        for cp in cps:
            cp.wait()

    plsc.subcore_barrier()

    for j in range(HID):
        @pl.when(s == j // 2)
        def _dw(acc=accs[j], part=parts[j]):
            pltpu.sync_copy(acc, part.at[c, 0])


@functools.partial(
    pl.kernel,
    out_type=[jax.ShapeDtypeStruct((NC, 1, NPAD), jnp.float32)
              for _ in range(HID)],
    mesh=_mesh,
    scratch_types=(
        [pltpu.VMEM_SHARED((NPAD,), jnp.float32) for _ in range(HID)] + [
            pltpu.VMEM((1, CHB), jnp.int32),
            pltpu.VMEM((1, CHB), jnp.int32),
            pltpu.VMEM((CHB,), jnp.float32),
            pltpu.VMEM((HID * CHB,), jnp.float32),
            pltpu.SemaphoreType.DMA,
            pltpu.SemaphoreType.DMA,
        ]),
    compiler_params=pltpu.CompilerParams(needs_layout_passes=False, use_tc_tiling_on_sc=False),
)
def _sc_hop(*args):
    _sc_hop_body(*args)


def _elu(x):
    return jnp.where(x > 0, x, jnp.exp(jnp.minimum(x, 0.0)) - 1.0)


def _enc_body(x_ref, idr_ref, e0_ref, e1_ref, e2_ref, dega_ref, degb_ref,
              w0_ref, b0_ref, wid_ref, bid_ref, wemb_ref, bemb_ref,
              lng_ref, lnb_ref, tw_ref,
              g0_ref, g1_ref, g2_ref, g3_ref, dinv_ref):
    h0 = _elu(jnp.dot(x_ref[...], w0_ref[...],
                      preferred_element_type=jnp.float32) + b0_ref[...])
    ide = _elu(jnp.dot(idr_ref[...], wid_ref[...],
                       preferred_element_type=jnp.float32) + bid_ref[...])
    cat = jnp.concatenate([e0_ref[...], e1_ref[...], e2_ref[...]], axis=1)
    ee = _elu(jnp.dot(cat, wemb_ref[...],
                      preferred_element_type=jnp.float32) + bemb_ref[...])
    h = jnp.concatenate([ide, h0, ee], axis=1)
    mu = jnp.mean(h, axis=-1, keepdims=True)
    var = jnp.mean((h - mu) ** 2, axis=-1, keepdims=True)
    hn = (h - mu) * jax.lax.rsqrt(var + EPS) * lng_ref[...] + lnb_ref[...]
    tw = tw_ref[...]
    g0_ref[...] = jnp.dot(hn, tw[0], preferred_element_type=jnp.float32)
    g1_ref[...] = jnp.dot(hn, tw[1], preferred_element_type=jnp.float32)
    g2_ref[...] = jnp.dot(hn, tw[2], preferred_element_type=jnp.float32)
    g3_ref[...] = jnp.dot(hn, tw[3], preferred_element_type=jnp.float32)
    deg = dega_ref[...] + degb_ref[...]
    dinv_ref[...] = jnp.where(
        deg > 0, jax.lax.rsqrt(jnp.maximum(deg, 1e-30)), 0.0)


def _encoder(x, idr, e0, e1, e2, dega, degb, params):
    D = IDE + HID + EMB * NCAT
    full = lambda shape: pl.BlockSpec(shape, lambda i: tuple(0 for _ in shape))
    row = lambda w: pl.BlockSpec((BN, w), lambda i: (i, 0))
    out_shapes = [jax.ShapeDtypeStruct((N, HID), jnp.float32) for _ in range(4)]
    out_shapes.append(jax.ShapeDtypeStruct((N, 1), jnp.float32))
    out_specs = [row(HID) for _ in range(4)] + [row(1)]
    return pl.pallas_call(
        _enc_body,
        grid=(GRID,),
        in_specs=[row(F_NUM), row(IDE), row(EMB), row(EMB), row(EMB),
                  row(1), row(1),
                  full((F_NUM, HID)), full((HID,)),
                  full((IDE, IDE)), full((IDE,)),
                  full((EMB * NCAT, EMB * NCAT)), full((EMB * NCAT,)),
                  full((D,)), full((D,)),
                  full((K + 1, D, HID))],
        out_specs=out_specs,
        out_shape=out_shapes,
    )(x, idr, e0, e1, e2, dega, degb,
      params['W0'], params['b0'], params['W_id'], params['b_id'],
      params['W_emb'], params['b_emb'], params['ln0_g'], params['ln0_b'],
      params['tag_W'])


CGRID = NPAD // CH


def _comb_body(*refs):
    p_refs = refs[:HID]
    gc_ref = refs[HID]
    out_refs = refs[HID + 1:]
    for j in range(HID):
        out_refs[j][...] = (p_refs[j][0, 0] + p_refs[j][1, 0]
                            + gc_ref[j, 0, :])


def _combine(parts, gcols):
    pspec = pl.BlockSpec((NC, 1, CH), lambda i: (0, 0, i))
    return pl.pallas_call(
        _comb_body,
        grid=(CGRID,),
        in_specs=[pspec] * HID + [pl.BlockSpec((HID, 1, CH),
                                               lambda i: (0, 0, i))],
        out_specs=[pl.BlockSpec((CH,), lambda i: (i,))] * HID,
        out_shape=[jax.ShapeDtypeStruct((NPAD,), jnp.float32)] * HID,
    )(*parts, gcols)


def _head_body(*refs):
    p_refs = refs[:HID]
    (gc_ref, tb_ref, lng_ref, lnb_ref, w1t_ref, b1_ref) = refs[HID:HID + 6]
    out_ref = refs[HID + 6]
    t = jnp.concatenate(
        [p_refs[j][...].sum(axis=0) for j in range(HID)], axis=0
    ) + gc_ref[:, 0, :]
    t = jnp.maximum(t + tb_ref[...][:, None], 0.0)
    mu = jnp.mean(t, axis=0, keepdims=True)
    var = jnp.mean((t - mu) ** 2, axis=0, keepdims=True)
    tn = ((t - mu) * jax.lax.rsqrt(var + EPS) * lng_ref[...][:, None]
          + lnb_ref[...][:, None])
    logits = jnp.dot(w1t_ref[...], tn,
                     preferred_element_type=jnp.float32) + b1_ref[...][:, None]
    m = jnp.max(logits, axis=0, keepdims=True)
    lse = m + jnp.log(jnp.sum(jnp.exp(logits - m), axis=0, keepdims=True))
    out_ref[...] = logits - lse


def _head(parts, g0cols, params):
    full = lambda shape: pl.BlockSpec(shape, lambda i: tuple(0 for _ in shape))
    pspec = pl.BlockSpec((NC, 1, CH), lambda i: (0, 0, i))
    return pl.pallas_call(
        _head_body,
        grid=(CGRID,),
        in_specs=[pspec] * HID + [
            pl.BlockSpec((HID, 1, CH), lambda i: (0, 0, i)),
            full((HID,)), full((HID,)), full((HID,)),
            full((NCLS, HID)), full((NCLS,))],
        out_specs=pl.BlockSpec((NCLS, CH), lambda i: (0, i)),
        out_shape=jax.ShapeDtypeStruct((NCLS, NPAD), jnp.float32),
    )(*parts, g0cols, params['tag_b'], params['ln1_g'], params['ln1_b'],
      params['W1'].T, params['b1'])


def kernel(x, edge_index, edge_weight, categories_value, params):
    src = edge_index[0]
    dst = edge_index[1]
    emb_cat = params['emb_tables'].reshape(NCAT * N, EMB)

    degp, idr, er = _sc_pre(dst, edge_weight,
                            categories_value.reshape(N * 4),
                            params['id_table'], emb_cat)

    g0, g1, g2, g3, dinv2 = _encoder(
        x, idr, er[0], er[1], er[2],
        degp[0, 0, :N][:, None], degp[1, 0, :N][:, None], params)
    dinv = dinv2[:, 0]

    norm = _sc_norm(src, dst, edge_weight, dinv)

    def cols(g):
        return jnp.pad(g.T, ((0, 0), (0, NPAD - N)))[:, None, :]

    g0c, g1c, g2c, g3c = cols(g0), cols(g1), cols(g2), cols(g3)
    tlist = [g3c[j, 0] for j in range(HID)]
    for gc in (g2c, g1c):
        parts = _sc_hop(*tlist, src, dst, norm)
        tlist = _combine(parts, gc)
    parts = _sc_hop(*tlist, src, dst, norm)
    out2 = _head(parts, g0c, params)
    return out2[:, :N].T
